# baseline (speedup 1.0000x reference)
import functools

import jax
import jax.numpy as jnp
from jax import lax
from jax.experimental import pallas as pl
from jax.experimental.pallas import tpu as pltpu
from jax.experimental.pallas import tpu_sc as plsc

BATCH = 16384
EMBED_DIM = 32
HIDDEN_DIM = 64
LANES = 128
PACK = LANES // EMBED_DIM

_CHUNK = 128
_RBLK = 32768
_TROW = 8


_GROUPS = _RBLK // 512


def _repack_body(tabT_ref, out_ref):
    x = tabT_ref[...].reshape(EMBED_DIM, _RBLK)
    for g in range(_GROUPS):
        stacked = jnp.concatenate(
            [x[:, g * 512 + s * LANES: g * 512 + (s + 1) * LANES]
             for s in range(PACK)], axis=0)
        out_ref[g * LANES:(g + 1) * LANES, :] = stacked.T


def _repack(tabT3, n_blocks):
    out_rows = n_blocks * _GROUPS * LANES
    return pl.pallas_call(
        _repack_body,
        grid=(n_blocks,),
        in_specs=[
            pl.BlockSpec((EMBED_DIM // _TROW, _TROW, _RBLK),
                         lambda b: (0, 0, b)),
        ],
        out_specs=pl.BlockSpec((_GROUPS * LANES, LANES), lambda b: (b, 0)),
        out_shape=jax.ShapeDtypeStruct((out_rows, LANES), jnp.float32),
    )(tabT3)


def _make_sc_gather(batch):
    info = plsc.get_sparse_core_info()
    nc, ns = info.num_cores, info.num_subcores
    nw = nc * ns
    b_per_w = batch // nw
    n_chunks = b_per_w // _CHUNK
    mesh = plsc.VectorSubcoreMesh(core_axis_name="c", subcore_axis_name="s")

    @functools.partial(
        pl.kernel,
        mesh=mesh,
        out_type=jax.ShapeDtypeStruct((batch, LANES), jnp.float32),
        scratch_types=[
            pltpu.VMEM((n_chunks, _CHUNK), jnp.int32),
            pltpu.VMEM((b_per_w, LANES), jnp.float32),
            pltpu.SemaphoreType.DMA,
        ],
    )
    def gather(idx_hbm, tab_hbm, out_hbm, idx_v, rows_v, sem):
        wid = lax.axis_index("s") * nc + lax.axis_index("c")
        base = wid * b_per_w
        row0 = wid * n_chunks
        pltpu.sync_copy(idx_hbm.at[pl.ds(row0, n_chunks)], idx_v)
        copies = []
        for j in range(n_chunks):
            copies.append(pltpu.async_copy(
                tab_hbm.at[idx_v.at[j]],
                rows_v.at[pl.ds(j * _CHUNK, _CHUNK)], sem))
        for cp in copies:
            cp.wait()
        pltpu.sync_copy(rows_v, out_hbm.at[pl.ds(base, b_per_w)])

    return gather


def _select_segment(rows, sel):
    out = None
    for s in range(PACK):
        seg = rows[:, s * EMBED_DIM:(s + 1) * EMBED_DIM]
        m = (sel == s).astype(jnp.float32)
        out = seg * m if out is None else out + seg * m
    return out


def _mlp_body(pu_ref, pi_ref, su_ref, si_ref, w1a_ref, w1b_ref, b1_ref,
              w2_ref, b2_ref, o_ref):
    u = _select_segment(pu_ref[...], su_ref[...])
    i = _select_segment(pi_ref[...], si_ref[...])
    h = (jnp.dot(u, w1a_ref[...], preferred_element_type=jnp.float32)
         + jnp.dot(i, w1b_ref[...], preferred_element_type=jnp.float32)
         + b1_ref[...])
    h = jnp.maximum(h, 0.0)
    o_ref[...] = jnp.sum(h * w2_ref[...], axis=1, keepdims=True) + b2_ref[...]


def _mlp(u_rows, i_rows, usel, isel, W1, b1, W2, b2):
    blk = 2048
    grid = BATCH // blk
    w1a = W1[:EMBED_DIM]
    w1b = W1[EMBED_DIM:]
    b1r = b1.reshape(1, HIDDEN_DIM)
    w2r = W2.reshape(1, HIDDEN_DIM)
    b2r = b2.reshape(1, 1)
    return pl.pallas_call(
        _mlp_body,
        grid=(grid,),
        in_specs=[
            pl.BlockSpec((blk, LANES), lambda b: (b, 0)),
            pl.BlockSpec((blk, LANES), lambda b: (b, 0)),
            pl.BlockSpec((blk, 1), lambda b: (b, 0)),
            pl.BlockSpec((blk, 1), lambda b: (b, 0)),
            pl.BlockSpec((EMBED_DIM, HIDDEN_DIM), lambda b: (0, 0)),
            pl.BlockSpec((EMBED_DIM, HIDDEN_DIM), lambda b: (0, 0)),
            pl.BlockSpec((1, HIDDEN_DIM), lambda b: (0, 0)),
            pl.BlockSpec((1, HIDDEN_DIM), lambda b: (0, 0)),
            pl.BlockSpec((1, 1), lambda b: (0, 0)),
        ],
        out_specs=pl.BlockSpec((blk, 1), lambda b: (b, 0)),
        out_shape=jax.ShapeDtypeStruct((BATCH, 1), jnp.float32),
    )(u_rows, i_rows, usel, isel, w1a, w1b, b1r, w2r, b2r)


def kernel(user_ids, item_ids, user_table, item_table, W1, b1, W2, b2):
    uids = user_ids.astype(jnp.int32)
    iids = item_ids.astype(jnp.int32)
    up = (((uids >> 9) << 7) + (uids & 127)).reshape(BATCH // _CHUNK, _CHUNK)
    ip = (((iids >> 9) << 7) + (iids & 127)).reshape(BATCH // _CHUNK, _CHUNK)
    usel = ((uids >> 7) & (PACK - 1)).reshape(BATCH, 1)
    isel = ((iids >> 7) & (PACK - 1)).reshape(BATCH, 1)
    n_rows = user_table.shape[0]
    n_blocks = -(-n_rows // _RBLK)
    utabT3 = user_table.T.reshape(EMBED_DIM // _TROW, _TROW, n_rows)
    itabT3 = item_table.T.reshape(EMBED_DIM // _TROW, _TROW, n_rows)
    gather = _make_sc_gather(BATCH)
    utab = _repack(utabT3, n_blocks)
    u_rows = gather(up, utab)
    itab = _repack(itabT3, n_blocks)
    i_rows = gather(ip, itab)
    return _mlp(u_rows, i_rows, usel, isel, W1, b1, W2, b2)


# speedup vs baseline: 1.0288x; 1.0288x over previous
import functools

import jax
import jax.numpy as jnp
from jax import lax
from jax.experimental import pallas as pl
from jax.experimental.pallas import tpu as pltpu
from jax.experimental.pallas import tpu_sc as plsc

BATCH = 16384
EMBED_DIM = 32
HIDDEN_DIM = 64
LANES = 128
PACK = LANES // EMBED_DIM

_CHUNK = 128
_RBLK = 32768
_TROW = 8

_GROUPS = _RBLK // 512


def _repack_body(utabT_ref, itabT_ref, uout_ref, iout_ref):
    for ref, out in ((utabT_ref, uout_ref), (itabT_ref, iout_ref)):
        x = ref[...].reshape(EMBED_DIM, _RBLK)
        for g in range(_GROUPS):
            stacked = jnp.concatenate(
                [x[:, g * 512 + s * LANES: g * 512 + (s + 1) * LANES]
                 for s in range(PACK)], axis=0)
            out[g * LANES:(g + 1) * LANES, :] = stacked.T


def _repack(utabT3, itabT3, n_blocks):
    out_rows = n_blocks * _GROUPS * LANES
    return pl.pallas_call(
        _repack_body,
        grid=(n_blocks,),
        in_specs=[
            pl.BlockSpec((EMBED_DIM // _TROW, _TROW, _RBLK),
                         lambda b: (0, 0, b)),
            pl.BlockSpec((EMBED_DIM // _TROW, _TROW, _RBLK),
                         lambda b: (0, 0, b)),
        ],
        out_specs=[
            pl.BlockSpec((_GROUPS * LANES, LANES), lambda b: (b, 0)),
            pl.BlockSpec((_GROUPS * LANES, LANES), lambda b: (b, 0)),
        ],
        out_shape=[
            jax.ShapeDtypeStruct((out_rows, LANES), jnp.float32),
            jax.ShapeDtypeStruct((out_rows, LANES), jnp.float32),
        ],
    )(utabT3, itabT3)


def _make_sc_gather(batch):
    info = plsc.get_sparse_core_info()
    nc, ns = info.num_cores, info.num_subcores
    nw = nc * ns
    b_per_w = batch // nw
    n_chunks = b_per_w // _CHUNK
    mesh = plsc.VectorSubcoreMesh(core_axis_name="c", subcore_axis_name="s")

    @functools.partial(
        pl.kernel,
        mesh=mesh,
        out_type=[
            jax.ShapeDtypeStruct((batch, LANES), jnp.float32),
            jax.ShapeDtypeStruct((batch, LANES), jnp.float32),
        ],
        scratch_types=[
            pltpu.VMEM((n_chunks, _CHUNK), jnp.int32),
            pltpu.VMEM((n_chunks, _CHUNK), jnp.int32),
            pltpu.VMEM((b_per_w, LANES), jnp.float32),
            pltpu.SemaphoreType.DMA,
        ],
    )
    def gather(uidx_hbm, iidx_hbm, utab_hbm, itab_hbm, uout_hbm, iout_hbm,
               uidx_v, iidx_v, rows_v, sem):
        wid = lax.axis_index("s") * nc + lax.axis_index("c")
        base = wid * b_per_w
        row0 = wid * n_chunks
        pltpu.sync_copy(uidx_hbm.at[pl.ds(row0, n_chunks)], uidx_v)
        pltpu.sync_copy(iidx_hbm.at[pl.ds(row0, n_chunks)], iidx_v)
        for idx_v, tab_hbm, out_hbm in ((uidx_v, utab_hbm, uout_hbm),
                                        (iidx_v, itab_hbm, iout_hbm)):
            copies = []
            for j in range(n_chunks):
                copies.append(pltpu.async_copy(
                    tab_hbm.at[idx_v.at[j]],
                    rows_v.at[pl.ds(j * _CHUNK, _CHUNK)], sem))
            for cp in copies:
                cp.wait()
            pltpu.sync_copy(rows_v, out_hbm.at[pl.ds(base, b_per_w)])

    return gather


def _select_segment(rows, sel):
    out = None
    for s in range(PACK):
        seg = rows[:, s * EMBED_DIM:(s + 1) * EMBED_DIM]
        m = (sel == s).astype(jnp.float32)
        out = seg * m if out is None else out + seg * m
    return out


def _mlp_body(pu_ref, pi_ref, su_ref, si_ref, w1a_ref, w1b_ref, b1_ref,
              w2_ref, b2_ref, o_ref):
    u = _select_segment(pu_ref[...], su_ref[...])
    i = _select_segment(pi_ref[...], si_ref[...])
    h = (jnp.dot(u, w1a_ref[...], preferred_element_type=jnp.float32)
         + jnp.dot(i, w1b_ref[...], preferred_element_type=jnp.float32)
         + b1_ref[...])
    h = jnp.maximum(h, 0.0)
    o_ref[...] = jnp.sum(h * w2_ref[...], axis=1, keepdims=True) + b2_ref[...]


def _mlp(u_rows, i_rows, usel, isel, W1, b1, W2, b2):
    blk = 2048
    grid = BATCH // blk
    w1a = W1[:EMBED_DIM]
    w1b = W1[EMBED_DIM:]
    b1r = b1.reshape(1, HIDDEN_DIM)
    w2r = W2.reshape(1, HIDDEN_DIM)
    b2r = b2.reshape(1, 1)
    return pl.pallas_call(
        _mlp_body,
        grid=(grid,),
        in_specs=[
            pl.BlockSpec((blk, LANES), lambda b: (b, 0)),
            pl.BlockSpec((blk, LANES), lambda b: (b, 0)),
            pl.BlockSpec((blk, 1), lambda b: (b, 0)),
            pl.BlockSpec((blk, 1), lambda b: (b, 0)),
            pl.BlockSpec((EMBED_DIM, HIDDEN_DIM), lambda b: (0, 0)),
            pl.BlockSpec((EMBED_DIM, HIDDEN_DIM), lambda b: (0, 0)),
            pl.BlockSpec((1, HIDDEN_DIM), lambda b: (0, 0)),
            pl.BlockSpec((1, HIDDEN_DIM), lambda b: (0, 0)),
            pl.BlockSpec((1, 1), lambda b: (0, 0)),
        ],
        out_specs=pl.BlockSpec((blk, 1), lambda b: (b, 0)),
        out_shape=jax.ShapeDtypeStruct((BATCH, 1), jnp.float32),
    )(u_rows, i_rows, usel, isel, w1a, w1b, b1r, w2r, b2r)


def kernel(user_ids, item_ids, user_table, item_table, W1, b1, W2, b2):
    uids = user_ids.astype(jnp.int32)
    iids = item_ids.astype(jnp.int32)
    up = (((uids >> 9) << 7) + (uids & 127)).reshape(BATCH // _CHUNK, _CHUNK)
    ip = (((iids >> 9) << 7) + (iids & 127)).reshape(BATCH // _CHUNK, _CHUNK)
    usel = ((uids >> 7) & (PACK - 1)).reshape(BATCH, 1)
    isel = ((iids >> 7) & (PACK - 1)).reshape(BATCH, 1)
    n_rows = user_table.shape[0]
    n_blocks = -(-n_rows // _RBLK)
    utabT3 = user_table.T.reshape(EMBED_DIM // _TROW, _TROW, n_rows)
    itabT3 = item_table.T.reshape(EMBED_DIM // _TROW, _TROW, n_rows)
    utab, itab = _repack(utabT3, itabT3, n_blocks)
    gather = _make_sc_gather(BATCH)
    u_rows, i_rows = gather(up, ip, utab, itab)
    return _mlp(u_rows, i_rows, usel, isel, W1, b1, W2, b2)
